# Initial kernel scaffold; baseline (speedup 1.0000x reference)
import jax
import jax.numpy as jnp
from jax.experimental import pallas as pl


def kernel(feat, edge_index, W1m, b1m, W1v, b1v, W2m, b2m, W2v, b2v):
    raise NotImplementedError("write your pallas kernel here")


# speedup vs baseline: 6.5944x; 6.5944x over previous
import functools

import jax
import jax.numpy as jnp
from jax import lax
from jax.experimental import pallas as pl
from jax.experimental.pallas import tpu as pltpu
from jax.experimental.pallas import tpu_sc as plsc

_N = 100000
_E = 3200000
_NC, _NS = 2, 16
_B = 128
_CB = 56
_NCHUNK = 28
_NBATCH = 25088
_EP = _NBATCH * _B
_NP = 100096
_STRIPE = _NP // (_NC * _NS)


def _mesh():
    return plsc.VectorSubcoreMesh(core_axis_name="c", subcore_axis_name="s")


def _deg_body(dst_hbm, ones_hbm, zcol_hbm, out_hbm, dacc, dbuf, ones_v, sem):
    c = lax.axis_index("c")
    t = lax.axis_index("s")
    wid = c * _NS + t
    r0 = t * _STRIPE
    pltpu.sync_copy(ones_hbm, ones_v)
    pltpu.sync_copy(zcol_hbm, dacc.at[pl.ds(r0, _STRIPE)])
    plsc.subcore_barrier()
    base = wid * (_NBATCH // (_NC * _NS))

    def chunk(ci, carry):
        b0 = base + ci * _CB
        pltpu.sync_copy(dst_hbm.at[pl.ds(b0, _CB)], dbuf)

        def batch(j, carry2):
            pltpu.sync_copy(ones_v, dacc.at[dbuf.at[j]], add=True)
            return carry2

        return lax.fori_loop(0, _CB, batch, carry)

    lax.fori_loop(0, (_NBATCH // (_NC * _NS)) // _CB, chunk, 0)
    plsc.subcore_barrier()
    pltpu.sync_copy(dacc.at[pl.ds(r0, _STRIPE)], out_hbm.at[c, pl.ds(r0, _STRIPE)])


def _degrees(dst2d, ones, zcol):
    return pl.kernel(
        _deg_body,
        out_type=jax.ShapeDtypeStruct((_NC, _NP, 1), jnp.float32),
        mesh=_mesh(),
        scratch_types=[
            pltpu.VMEM_SHARED((_NP, 1), jnp.float32),
            pltpu.VMEM((_CB, _B), jnp.int32),
            pltpu.VMEM((_B, 1), jnp.float32),
            pltpu.SemaphoreType.DMA,
        ],
        compiler_params=pltpu.CompilerParams(use_tc_tiling_on_sc=False),
    )(dst2d, ones, zcol)


def _agg_body(msgv_hbm, src_hbm, dst_hbm, zrow_hbm, out_hbm,
              acc, sbuf, dbuf, sidx, rows, gsem):
    c = lax.axis_index("c")
    t = lax.axis_index("s")
    r0 = t * _STRIPE
    base = t * (_NBATCH // _NS)

    for sl in range(4):
        s_glob = c * 4 + sl
        pltpu.sync_copy(zrow_hbm, acc.at[pl.ds(r0, _STRIPE)])
        plsc.subcore_barrier()

        def chunk(ci, carry, s_glob=s_glob):
            b0 = base + ci * _CB
            pltpu.sync_copy(src_hbm.at[pl.ds(b0, _CB)], sbuf)
            pltpu.sync_copy(dst_hbm.at[pl.ds(b0, _CB)], dbuf)

            def mkidx(j, carry2):
                for i in range(_B // 16):
                    v = sbuf[j, pl.ds(i * 16, 16)]
                    sidx[j, pl.ds(i * 16, 16)] = v * 8 + s_glob
                return carry2

            lax.fori_loop(0, _CB, mkidx, 0)

            pltpu.async_copy(msgv_hbm.at[sidx.at[0]], rows.at[0], gsem)

            def batch(j, carry2):
                nxt = j + 1

                @pl.when(nxt < _CB)
                def _():
                    pltpu.async_copy(msgv_hbm.at[sidx.at[nxt]],
                                     rows.at[nxt % 2], gsem)

                pltpu.make_async_copy(msgv_hbm.at[sidx.at[j]],
                                      rows.at[j % 2], gsem).wait()
                pltpu.sync_copy(rows.at[j % 2], acc.at[dbuf.at[j]], add=True)
                return carry2

            lax.fori_loop(0, _CB, batch, 0)
            return carry

        lax.fori_loop(0, _NCHUNK, chunk, 0)
        plsc.subcore_barrier()
        pltpu.sync_copy(acc.at[pl.ds(r0, _STRIPE)],
                        out_hbm.at[s_glob, pl.ds(r0, _STRIPE)])
        plsc.subcore_barrier()


def _aggregate(msgv, src2d, dst2d, zrow):
    return pl.kernel(
        _agg_body,
        out_type=jax.ShapeDtypeStruct((8, _NP, 16), jnp.float32),
        mesh=_mesh(),
        scratch_types=[
            pltpu.VMEM_SHARED((_NP, 16), jnp.float32),
            pltpu.VMEM((_CB, _B), jnp.int32),
            pltpu.VMEM((_CB, _B), jnp.int32),
            pltpu.VMEM((_CB, _B), jnp.int32),
            pltpu.VMEM((2, _B, 16), jnp.float32),
            pltpu.SemaphoreType.DMA,
        ],
        compiler_params=pltpu.CompilerParams(use_tc_tiling_on_sc=False),
    )(msgv, src2d, dst2d, zrow)


_R = 512
_GRID = (_N + _R - 1) // _R


def _layer1_body(feat_ref, w1m_ref, b1m_ref, w1v_ref, b1v_ref, deg_ref,
                 msg_ref, norm_ref):
    x = feat_ref[...]
    m = jnp.maximum(
        jnp.dot(x, w1m_ref[...], preferred_element_type=jnp.float32)
        + b1m_ref[...], 0.0)
    v = jnp.maximum(
        jnp.dot(x, w1v_ref[...], preferred_element_type=jnp.float32)
        + b1v_ref[...], 0.0)
    d = deg_ref[0, :] + deg_ref[1, :]
    norm = lax.rsqrt(jnp.maximum(d, 1.0))[:, None]
    att = jnp.exp(-v)
    msg_ref[...] = jnp.concatenate(
        [m * att * norm, v * (att * att) * (norm * norm)], axis=1)
    norm_ref[...] = norm


def _layer1(feat, w1m, b1m, w1v, b1v, deg2):
    return pl.pallas_call(
        _layer1_body,
        grid=(_GRID,),
        in_specs=[
            pl.BlockSpec((_R, 128), lambda i: (i, 0)),
            pl.BlockSpec((128, 64), lambda i: (0, 0)),
            pl.BlockSpec((1, 64), lambda i: (0, 0)),
            pl.BlockSpec((128, 64), lambda i: (0, 0)),
            pl.BlockSpec((1, 64), lambda i: (0, 0)),
            pl.BlockSpec((2, _R), lambda i: (0, i)),
        ],
        out_specs=[
            pl.BlockSpec((_R, 128), lambda i: (i, 0)),
            pl.BlockSpec((_R, 1), lambda i: (i, 0)),
        ],
        out_shape=[
            jax.ShapeDtypeStruct((_N, 128), jnp.float32),
            jax.ShapeDtypeStruct((_N, 1), jnp.float32),
        ],
    )(feat, w1m, b1m, w1v, b1v, deg2)


def _layer2_body(agg_ref, norm_ref, w2m_ref, b2m_ref, w2v_ref, b2v_ref,
                 msg_ref):
    norm = norm_ref[...]
    mean_in = agg_ref[:, :64] * norm
    var_in = agg_ref[:, 64:] * (norm * norm)
    m = jnp.dot(mean_in, w2m_ref[...],
                preferred_element_type=jnp.float32) + b2m_ref[...]
    v = jnp.dot(var_in, w2v_ref[...],
                preferred_element_type=jnp.float32) + b2v_ref[...]
    att = jnp.exp(-v)
    msg_ref[...] = jnp.concatenate(
        [m * att * norm, v * (att * att) * (norm * norm)], axis=1)


def _layer2(agg1, norm, w2m, b2m, w2v, b2v):
    return pl.pallas_call(
        _layer2_body,
        grid=(_GRID,),
        in_specs=[
            pl.BlockSpec((_R, 128), lambda i: (i, 0)),
            pl.BlockSpec((_R, 1), lambda i: (i, 0)),
            pl.BlockSpec((64, 64), lambda i: (0, 0)),
            pl.BlockSpec((1, 64), lambda i: (0, 0)),
            pl.BlockSpec((64, 64), lambda i: (0, 0)),
            pl.BlockSpec((1, 64), lambda i: (0, 0)),
        ],
        out_specs=pl.BlockSpec((_R, 128), lambda i: (i, 0)),
        out_shape=jax.ShapeDtypeStruct((_N, 128), jnp.float32),
    )(agg1, norm, w2m, b2m, w2v, b2v)


def _final_body(agg_ref, norm_ref, eps_ref, z_ref):
    norm = norm_ref[...]
    mean = agg_ref[:, :64] * norm
    var = agg_ref[:, 64:] * (norm * norm)
    z_ref[...] = eps_ref[...] * jnp.sqrt(var + 1e-8) + mean


def _final(agg2, norm, eps):
    return pl.pallas_call(
        _final_body,
        grid=(_GRID,),
        in_specs=[
            pl.BlockSpec((_R, 128), lambda i: (i, 0)),
            pl.BlockSpec((_R, 1), lambda i: (i, 0)),
            pl.BlockSpec((_R, 64), lambda i: (i, 0)),
        ],
        out_specs=pl.BlockSpec((_R, 64), lambda i: (i, 0)),
        out_shape=jax.ShapeDtypeStruct((_N, 64), jnp.float32),
    )(agg2, norm, eps)


def kernel(feat, edge_index, W1m, b1m, W1v, b1v, W2m, b2m, W2v, b2v):
    src = edge_index[0]
    dst = edge_index[1]
    pad = _EP - _E
    src2d = jnp.concatenate(
        [src, jnp.zeros((pad,), jnp.int32)]).reshape(_NBATCH, _B)
    dst2d = jnp.concatenate(
        [dst, jnp.full((pad,), _N, jnp.int32)]).reshape(_NBATCH, _B)

    ones = jnp.ones((_B, 1), jnp.float32)
    zcol = jnp.zeros((_STRIPE, 1), jnp.float32)
    zrow = jnp.zeros((_STRIPE, 16), jnp.float32)

    deg2 = _degrees(dst2d, ones, zcol)
    deg2 = deg2[:, :_N, 0]

    msg1, norm = _layer1(feat, W1m, b1m.reshape(1, 64),
                         W1v, b1v.reshape(1, 64), deg2)
    agg1 = _aggregate(msg1.reshape(8 * _N, 16), src2d, dst2d, zrow)
    agg1 = agg1[:, :_N, :].transpose(1, 0, 2).reshape(_N, 128)

    msg2 = _layer2(agg1, norm, W2m, b2m.reshape(1, 64),
                   W2v, b2v.reshape(1, 64))
    agg2 = _aggregate(msg2.reshape(8 * _N, 16), src2d, dst2d, zrow)
    agg2 = agg2[:, :_N, :].transpose(1, 0, 2).reshape(_N, 128)

    eps = jax.random.normal(jax.random.key(42), (_N, 64), dtype=jnp.float32)
    return _final(agg2, norm, eps)


# speedup vs baseline: 7.5747x; 1.1487x over previous
import functools

import jax
import jax.numpy as jnp
from jax import lax
from jax.experimental import pallas as pl
from jax.experimental.pallas import tpu as pltpu
from jax.experimental.pallas import tpu_sc as plsc

_N = 100000
_E = 3200000
_NC, _NS = 2, 16
_B = 128
_CB = 56
_NCHUNK = 28
_NBATCH = 25088
_EP = _NBATCH * _B
_NP = 100096
_STRIPE = _NP // (_NC * _NS)


def _mesh():
    return plsc.VectorSubcoreMesh(core_axis_name="c", subcore_axis_name="s")


def _deg_body(dst_hbm, ones_hbm, zcol_hbm, out_hbm, dacc, dbuf, ones_v, sem):
    c = lax.axis_index("c")
    t = lax.axis_index("s")
    wid = c * _NS + t
    r0 = t * _STRIPE
    pltpu.sync_copy(ones_hbm, ones_v)
    pltpu.sync_copy(zcol_hbm, dacc.at[pl.ds(r0, _STRIPE)])
    plsc.subcore_barrier()
    base = wid * (_NBATCH // (_NC * _NS))

    def chunk(ci, carry):
        b0 = base + ci * _CB
        pltpu.sync_copy(dst_hbm.at[pl.ds(b0, _CB)], dbuf)

        def batch(j, carry2):
            pltpu.sync_copy(ones_v, dacc.at[dbuf.at[j]], add=True)
            return carry2

        return lax.fori_loop(0, _CB, batch, carry)

    lax.fori_loop(0, (_NBATCH // (_NC * _NS)) // _CB, chunk, 0)
    plsc.subcore_barrier()
    pltpu.sync_copy(dacc.at[pl.ds(r0, _STRIPE)], out_hbm.at[c, pl.ds(r0, _STRIPE)])


def _degrees(dst2d, ones, zcol):
    return pl.kernel(
        _deg_body,
        out_type=jax.ShapeDtypeStruct((_NC, _NP, 1), jnp.float32),
        mesh=_mesh(),
        scratch_types=[
            pltpu.VMEM_SHARED((_NP, 1), jnp.float32),
            pltpu.VMEM((_CB, _B), jnp.int32),
            pltpu.VMEM((_B, 1), jnp.float32),
            pltpu.SemaphoreType.DMA,
        ],
        compiler_params=pltpu.CompilerParams(use_tc_tiling_on_sc=False),
    )(dst2d, ones, zcol)


def _agg_body(msgv_hbm, src_hbm, dst_hbm, zrow_hbm, out_hbm,
              acc, sbuf, dbuf, sidx, rows, gsem, ssem):
    c = lax.axis_index("c")
    t = lax.axis_index("s")
    r0 = t * _STRIPE
    base = t * (_NBATCH // _NS)

    for sl in range(4):
        s_glob = c * 4 + sl
        pltpu.sync_copy(zrow_hbm, acc.at[pl.ds(r0, _STRIPE)])
        plsc.subcore_barrier()

        def chunk(ci, carry, s_glob=s_glob):
            b0 = base + ci * _CB
            pltpu.sync_copy(src_hbm.at[pl.ds(b0, _CB)], sbuf)
            pltpu.sync_copy(dst_hbm.at[pl.ds(b0, _CB)], dbuf)

            def mkidx(j, carry2):
                for i in range(_B // 16):
                    v = sbuf[j, pl.ds(i * 16, 16)]
                    sidx[j, pl.ds(i * 16, 16)] = v * 8 + s_glob
                return carry2

            lax.fori_loop(0, _CB, mkidx, 0)

            pltpu.async_copy(msgv_hbm.at[sidx.at[0]], rows.at[0], gsem)
            pltpu.async_copy(msgv_hbm.at[sidx.at[1]], rows.at[1], gsem)

            def batch(j, carry2):
                @pl.when(j >= 2)
                def _():
                    pltpu.make_async_copy(rows.at[(j - 2) % 4],
                                          acc.at[dbuf.at[j - 2]], ssem).wait()

                @pl.when(j + 2 < _CB)
                def _():
                    pltpu.async_copy(msgv_hbm.at[sidx.at[j + 2]],
                                     rows.at[(j + 2) % 4], gsem)

                pltpu.make_async_copy(msgv_hbm.at[sidx.at[j]],
                                      rows.at[j % 4], gsem).wait()
                pltpu.async_copy(rows.at[j % 4], acc.at[dbuf.at[j]], ssem,
                                 add=True)
                return carry2

            lax.fori_loop(0, _CB, batch, 0)
            pltpu.make_async_copy(rows.at[(_CB - 2) % 4],
                                  acc.at[dbuf.at[_CB - 2]], ssem).wait()
            pltpu.make_async_copy(rows.at[(_CB - 1) % 4],
                                  acc.at[dbuf.at[_CB - 1]], ssem).wait()
            return carry

        lax.fori_loop(0, _NCHUNK, chunk, 0)
        plsc.subcore_barrier()
        pltpu.sync_copy(acc.at[pl.ds(r0, _STRIPE)],
                        out_hbm.at[s_glob, pl.ds(r0, _STRIPE)])
        plsc.subcore_barrier()


def _aggregate(msgv, src2d, dst2d, zrow):
    return pl.kernel(
        _agg_body,
        out_type=jax.ShapeDtypeStruct((8, _NP, 16), jnp.float32),
        mesh=_mesh(),
        scratch_types=[
            pltpu.VMEM_SHARED((_NP, 16), jnp.float32),
            pltpu.VMEM((_CB, _B), jnp.int32),
            pltpu.VMEM((_CB, _B), jnp.int32),
            pltpu.VMEM((_CB, _B), jnp.int32),
            pltpu.VMEM((4, _B, 16), jnp.float32),
            pltpu.SemaphoreType.DMA,
            pltpu.SemaphoreType.DMA,
        ],
        compiler_params=pltpu.CompilerParams(use_tc_tiling_on_sc=False),
    )(msgv, src2d, dst2d, zrow)


_R = 512
_GRID = (_N + _R - 1) // _R


def _layer1_body(feat_ref, w1m_ref, b1m_ref, w1v_ref, b1v_ref, deg_ref,
                 msg_ref, norm_ref):
    x = feat_ref[...]
    m = jnp.maximum(
        jnp.dot(x, w1m_ref[...], preferred_element_type=jnp.float32)
        + b1m_ref[...], 0.0)
    v = jnp.maximum(
        jnp.dot(x, w1v_ref[...], preferred_element_type=jnp.float32)
        + b1v_ref[...], 0.0)
    d = deg_ref[0, :] + deg_ref[1, :]
    norm = lax.rsqrt(jnp.maximum(d, 1.0))[:, None]
    att = jnp.exp(-v)
    msg_ref[...] = jnp.concatenate(
        [m * att * norm, v * (att * att) * (norm * norm)], axis=1)
    norm_ref[...] = norm


def _layer1(feat, w1m, b1m, w1v, b1v, deg2):
    return pl.pallas_call(
        _layer1_body,
        grid=(_GRID,),
        in_specs=[
            pl.BlockSpec((_R, 128), lambda i: (i, 0)),
            pl.BlockSpec((128, 64), lambda i: (0, 0)),
            pl.BlockSpec((1, 64), lambda i: (0, 0)),
            pl.BlockSpec((128, 64), lambda i: (0, 0)),
            pl.BlockSpec((1, 64), lambda i: (0, 0)),
            pl.BlockSpec((2, _R), lambda i: (0, i)),
        ],
        out_specs=[
            pl.BlockSpec((_R, 128), lambda i: (i, 0)),
            pl.BlockSpec((_R, 1), lambda i: (i, 0)),
        ],
        out_shape=[
            jax.ShapeDtypeStruct((_N, 128), jnp.float32),
            jax.ShapeDtypeStruct((_N, 1), jnp.float32),
        ],
    )(feat, w1m, b1m, w1v, b1v, deg2)


def _layer2_body(agg_ref, norm_ref, w2m_ref, b2m_ref, w2v_ref, b2v_ref,
                 msg_ref):
    norm = norm_ref[...]
    mean_in = agg_ref[:, :64] * norm
    var_in = agg_ref[:, 64:] * (norm * norm)
    m = jnp.dot(mean_in, w2m_ref[...],
                preferred_element_type=jnp.float32) + b2m_ref[...]
    v = jnp.dot(var_in, w2v_ref[...],
                preferred_element_type=jnp.float32) + b2v_ref[...]
    att = jnp.exp(-v)
    msg_ref[...] = jnp.concatenate(
        [m * att * norm, v * (att * att) * (norm * norm)], axis=1)


def _layer2(agg1, norm, w2m, b2m, w2v, b2v):
    return pl.pallas_call(
        _layer2_body,
        grid=(_GRID,),
        in_specs=[
            pl.BlockSpec((_R, 128), lambda i: (i, 0)),
            pl.BlockSpec((_R, 1), lambda i: (i, 0)),
            pl.BlockSpec((64, 64), lambda i: (0, 0)),
            pl.BlockSpec((1, 64), lambda i: (0, 0)),
            pl.BlockSpec((64, 64), lambda i: (0, 0)),
            pl.BlockSpec((1, 64), lambda i: (0, 0)),
        ],
        out_specs=pl.BlockSpec((_R, 128), lambda i: (i, 0)),
        out_shape=jax.ShapeDtypeStruct((_N, 128), jnp.float32),
    )(agg1, norm, w2m, b2m, w2v, b2v)


def _final_body(agg_ref, norm_ref, eps_ref, z_ref):
    norm = norm_ref[...]
    mean = agg_ref[:, :64] * norm
    var = agg_ref[:, 64:] * (norm * norm)
    z_ref[...] = eps_ref[...] * jnp.sqrt(var + 1e-8) + mean


def _final(agg2, norm, eps):
    return pl.pallas_call(
        _final_body,
        grid=(_GRID,),
        in_specs=[
            pl.BlockSpec((_R, 128), lambda i: (i, 0)),
            pl.BlockSpec((_R, 1), lambda i: (i, 0)),
            pl.BlockSpec((_R, 64), lambda i: (i, 0)),
        ],
        out_specs=pl.BlockSpec((_R, 64), lambda i: (i, 0)),
        out_shape=jax.ShapeDtypeStruct((_N, 64), jnp.float32),
    )(agg2, norm, eps)


def kernel(feat, edge_index, W1m, b1m, W1v, b1v, W2m, b2m, W2v, b2v):
    src = edge_index[0]
    dst = edge_index[1]
    pad = _EP - _E
    src2d = jnp.concatenate(
        [src, jnp.zeros((pad,), jnp.int32)]).reshape(_NBATCH, _B)
    dst2d = jnp.concatenate(
        [dst, jnp.full((pad,), _N, jnp.int32)]).reshape(_NBATCH, _B)

    ones = jnp.ones((_B, 1), jnp.float32)
    zcol = jnp.zeros((_STRIPE, 1), jnp.float32)
    zrow = jnp.zeros((_STRIPE, 16), jnp.float32)

    deg2 = _degrees(dst2d, ones, zcol)
    deg2 = deg2[:, :_N, 0]

    msg1, norm = _layer1(feat, W1m, b1m.reshape(1, 64),
                         W1v, b1v.reshape(1, 64), deg2)
    agg1 = _aggregate(msg1.reshape(8 * _N, 16), src2d, dst2d, zrow)
    agg1 = agg1[:, :_N, :].transpose(1, 0, 2).reshape(_N, 128)

    msg2 = _layer2(agg1, norm, W2m, b2m.reshape(1, 64),
                   W2v, b2v.reshape(1, 64))
    agg2 = _aggregate(msg2.reshape(8 * _N, 16), src2d, dst2d, zrow)
    agg2 = agg2[:, :_N, :].transpose(1, 0, 2).reshape(_N, 128)

    eps = jax.random.normal(jax.random.key(42), (_N, 64), dtype=jnp.float32)
    return _final(agg2, norm, eps)
